# baseline (speedup 1.0000x reference)
import math

import jax
import jax.numpy as jnp
from jax.experimental import pallas as pl
from jax.experimental.pallas import tpu as pltpu

DIM = 768
H = 8
HD = DIM // H
NC = 1000
NS = 50
RING = 4
SPB = 2
F32 = jnp.float32

_PK_BQ = 0
_PK_BO = 1
_PK_KEB1 = 2
_PK_KEG1 = 3
_PK_KEBE1 = 4
_PK_FUB1 = 5
_PK_FUG = 6
_PK_FUBE = 7
_PK_SELB = 8
_PKROWS = 9


def _body(x_hbm, pack_hbm, selW_hbm, skb_hbm,
          concepts_hbm, wq_hbm, wk_hbm, wv_hbm, wo_hbm,
          keW1_hbm, keW2_hbm, fuW1_hbm, fuW2_hbm, skW1_hbm, skW2_hbm,
          out_ref, sem_ref, acc_ref, sc_ref, w1buf, w2buf, sems,
          cbuf, wq_s, wk_s, wv_s, wo_s, keW1_s, keW2_s, fuW1_s, fuW2_s,
          csems, x_ref, pack_ref, selW_ref, skb_ref, tsems):
    k = pl.program_id(0)

    tcopies = [
        pltpu.make_async_copy(x_hbm, x_ref, tsems.at[0]),
        pltpu.make_async_copy(pack_hbm, pack_ref, tsems.at[1]),
        pltpu.make_async_copy(selW_hbm, selW_ref, tsems.at[2]),
        pltpu.make_async_copy(skb_hbm, skb_ref, tsems.at[3]),
    ]

    def _issue(skill, slot):
        pltpu.make_async_copy(skW1_hbm.at[skill], w1buf.at[slot],
                              sems.at[slot, 0]).start()
        pltpu.make_async_copy(skW2_hbm.at[skill], w2buf.at[slot],
                              sems.at[slot, 1]).start()

    ccopies = [
        pltpu.make_async_copy(concepts_hbm, cbuf, csems.at[0]),
        pltpu.make_async_copy(wq_hbm, wq_s, csems.at[1]),
        pltpu.make_async_copy(wk_hbm, wk_s, csems.at[2]),
        pltpu.make_async_copy(wv_hbm, wv_s, csems.at[3]),
        pltpu.make_async_copy(wo_hbm, wo_s, csems.at[4]),
        pltpu.make_async_copy(keW1_hbm, keW1_s, csems.at[5]),
        pltpu.make_async_copy(keW2_hbm, keW2_s, csems.at[6]),
        pltpu.make_async_copy(fuW1_hbm.at[pl.ds(2 * DIM, 2 * DIM)], fuW1_s,
                              csems.at[7]),
        pltpu.make_async_copy(fuW2_hbm, fuW2_s, csems.at[8]),
    ]

    @pl.when(k == 0)
    def _prologue():
        for c in tcopies:
            c.start()
        for j in range(RING):
            _issue(jnp.int32(j), jnp.int32(j))

    _sched = [(0, [0]), (1, [1, 2]), (2, [3, 4]), (3, [5]), (4, [6]),
              (5, [7]), (6, [8])]
    for _step, _idxs in _sched:
        @pl.when(k == _step)
        def _issue_consts(_idxs=_idxs):
            for j in _idxs:
                ccopies[j].start()

    @pl.when(k == 0)
    def _scores():
        for c in tcopies:
            c.wait()
        selb = pack_ref[_PK_SELB:_PK_SELB + 1, :NS]
        sl = jnp.dot(x_ref[...], selW_ref[...], preferred_element_type=F32) + selb
        sl = sl - jnp.max(sl, axis=-1, keepdims=True)
        e = jnp.exp(sl)
        sc_ref[...] = e / jnp.sum(e, axis=-1, keepdims=True)

    @pl.when(k == 7)
    def _init():
        for c in ccopies[:7]:
            c.wait()
        bq = pack_ref[_PK_BQ:_PK_BQ + 1, :DIM]
        bv = pack_ref[_PK_BQ:_PK_BQ + 1, DIM:]
        bo = pack_ref[_PK_BO:_PK_BO + 1, :DIM]
        keb2 = pack_ref[_PK_BO:_PK_BO + 1, DIM:]
        x = x_ref[...]
        q = jnp.dot(x, wq_s[...], preferred_element_type=F32) + bq
        rows = jax.lax.broadcasted_iota(jnp.int32, (H, DIM), 0)
        cols = jax.lax.broadcasted_iota(jnp.int32, (H, DIM), 1)
        maskf = (cols // HD == rows).astype(F32)
        q8 = maskf * q
        t = jax.lax.dot_general(wk_s[...], q8, (((1,), (1,)), ((), ())),
                                preferred_element_type=F32)
        logits = jnp.dot(cbuf[...], t,
                         preferred_element_type=F32) * (1.0 / math.sqrt(HD))
        m = jnp.max(logits, axis=0, keepdims=True)
        ew = jnp.exp(logits - m)
        w = ew / jnp.sum(ew, axis=0, keepdims=True)
        u = jax.lax.dot_general(w, cbuf[...], (((0,), (0,)), ((), ())),
                                preferred_element_type=F32)
        p = jnp.dot(u, wv_s[...], preferred_element_type=F32)
        o = jnp.sum(p * maskf, axis=0, keepdims=True) + bv
        attended = jnp.dot(o, wo_s[...], preferred_element_type=F32) + bo
        combined = x + attended
        keb1 = pack_ref[_PK_KEB1:_PK_KEB1 + 1, :]
        keg1 = pack_ref[_PK_KEG1:_PK_KEG1 + 1, :]
        kebe1 = pack_ref[_PK_KEBE1:_PK_KEBE1 + 1, :]
        y = jnp.dot(combined, keW1_s[...], preferred_element_type=F32) + keb1
        mu = jnp.mean(y, axis=-1, keepdims=True)
        var = jnp.mean((y - mu) * (y - mu), axis=-1, keepdims=True)
        yn = (y - mu) / jnp.sqrt(var + 1e-5) * keg1 + kebe1
        h1 = jnp.maximum(yn, 0.0)
        sem_ref[...] = jnp.dot(h1, keW2_s[...], preferred_element_type=F32) + keb2

    s_all = sc_ref[...]
    lane = jax.lax.broadcasted_iota(jnp.int32, (1, NS), 1)
    base = jax.lax.rem(k, RING // SPB) * SPB
    contrib = None
    for i in range(SPB):
        skill = SPB * k + i
        slot = base + i
        pltpu.make_async_copy(skW1_hbm.at[skill], w1buf.at[slot],
                              sems.at[slot, 0]).wait()
        pltpu.make_async_copy(skW2_hbm.at[skill], w2buf.at[slot],
                              sems.at[slot, 1]).wait()
        sk = jnp.sum(jnp.where(lane == skill, s_all, 0.0))
        b1 = skb_ref[pl.ds(skill, 1), :]
        b2 = skb_ref[pl.ds(NS + skill, 1), :]
        hk = jnp.maximum(
            jnp.dot(x_ref[...], w1buf[slot], preferred_element_type=F32) + b1, 0.0)
        outk = jnp.dot(hk, w2buf[slot], preferred_element_type=F32) + b2
        c = sk * outk
        contrib = c if contrib is None else contrib + c

    @pl.when(k == 0)
    def _first():
        acc_ref[...] = contrib

    @pl.when(k > 0)
    def _rest():
        acc_ref[...] = acc_ref[...] + contrib

    @pl.when(SPB * k + RING < NS)
    def _refill():
        for i in range(SPB):
            _issue(SPB * k + RING + i, base + i)

    @pl.when(k == NS // SPB - 1)
    def _fuse():
        for c in ccopies[7:]:
            c.wait()
        fub1 = pack_ref[_PK_FUB1:_PK_FUB1 + 1, :]
        fug = pack_ref[_PK_FUG:_PK_FUG + 1, :]
        fube = pack_ref[_PK_FUBE:_PK_FUBE + 1, :]
        fub2 = pack_ref[_PK_SELB:_PK_SELB + 1, DIM:]
        cat = jnp.concatenate([sem_ref[...], acc_ref[...]], axis=-1)
        y = jnp.dot(cat, fuW1_s[...], preferred_element_type=F32) + fub1
        mu = jnp.mean(y, axis=-1, keepdims=True)
        var = jnp.mean((y - mu) * (y - mu), axis=-1, keepdims=True)
        yn = (y - mu) / jnp.sqrt(var + 1e-5) * fug + fube
        fh = jnp.maximum(yn, 0.0)
        out_ref[...] = jnp.dot(fh, fuW2_s[...], preferred_element_type=F32) + fub2


def _const2d(shape):
    return pl.BlockSpec(shape, lambda k: (0, 0))


_HBM = pl.BlockSpec(memory_space=pltpu.MemorySpace.HBM)


@jax.jit
def kernel(x, concepts, Wq, bq, Wk, bk, Wv, bv, Wo, bo, keW1, keb1, keg1,
           kebe1, keW2, keb2, selW, selb, skW1, skb1, skW2, skb2,
           fuW1, fub1, fug, fube, fuW2, fub2):
    d = DIM
    z = jnp.zeros((2 * d,), F32)
    pack = jnp.stack([
        jnp.concatenate([bq, bv]),
        jnp.concatenate([bo, keb2]),
        keb1, keg1, kebe1, fub1, fug, fube,
        jnp.concatenate([selb, z[:d - NS], fub2]),
    ])
    skb = jnp.concatenate([skb1, skb2], axis=0)
    in_specs = [
        _HBM,
        _HBM,
        _HBM,
        _HBM,
        _HBM, _HBM, _HBM, _HBM, _HBM,
        _HBM, _HBM, _HBM, _HBM,
        _HBM, _HBM,
    ]
    out = pl.pallas_call(
        _body,
        grid=(NS // SPB,),
        in_specs=in_specs,
        out_specs=_const2d((1, d)),
        out_shape=jax.ShapeDtypeStruct((1, d), F32),
        scratch_shapes=[
            pltpu.VMEM((1, d), F32),
            pltpu.VMEM((1, d), F32),
            pltpu.VMEM((1, NS), F32),
            pltpu.VMEM((RING, d, d), F32),
            pltpu.VMEM((RING, d, d), F32),
            pltpu.SemaphoreType.DMA((RING, 2)),
            pltpu.VMEM((NC, d), F32),
            pltpu.VMEM((d, d), F32),
            pltpu.VMEM((d, d), F32),
            pltpu.VMEM((d, d), F32),
            pltpu.VMEM((d, d), F32),
            pltpu.VMEM((d, 2 * d), F32),
            pltpu.VMEM((2 * d, d), F32),
            pltpu.VMEM((2 * d, 2 * d), F32),
            pltpu.VMEM((2 * d, d), F32),
            pltpu.SemaphoreType.DMA((9,)),
            pltpu.VMEM((1, d), F32),
            pltpu.VMEM((_PKROWS, 2 * d), F32),
            pltpu.VMEM((d, NS), F32),
            pltpu.VMEM((2 * NS, d), F32),
            pltpu.SemaphoreType.DMA((4,)),
        ],
        compiler_params=pltpu.CompilerParams(
            dimension_semantics=("arbitrary",),
            vmem_limit_bytes=67108864,
        ),
    )(x, pack, selW, skb,
      concepts, Wq, Wk, Wv, Wo, keW1, keW2, fuW1, fuW2, skW1, skW2)
    return out


# speedup vs baseline: 1.0158x; 1.0158x over previous
import math

import jax
import jax.numpy as jnp
from jax.experimental import pallas as pl
from jax.experimental.pallas import tpu as pltpu

DIM = 768
H = 8
HD = DIM // H
NC = 1000
NS = 50
RING = 4
SPB = 2
F32 = jnp.float32

_PK_BQ = 0
_PK_BO = 1
_PK_KEB1 = 2
_PK_KEG1 = 3
_PK_KEBE1 = 4
_PK_FUB1 = 5
_PK_FUG = 6
_PK_FUBE = 7
_PK_SELB = 8
_PKROWS = 9


def _body(x_ref, pack_ref, selW_ref, skb_ref,
          concepts_hbm, wq_hbm, wk_hbm, wv_hbm, wo_hbm,
          keW1_hbm, keW2_hbm, fuW1_hbm, fuW2_hbm, skW1_hbm, skW2_hbm,
          out_ref, sem_ref, acc_ref, sc_ref, w1buf, w2buf, sems,
          cbuf, wq_s, wk_s, wv_s, wo_s, keW1_s, keW2_s, fuW1_s, fuW2_s,
          csems):
    k = pl.program_id(0)
    x = x_ref[...]

    def _issue(skill, slot):
        pltpu.make_async_copy(skW1_hbm.at[skill], w1buf.at[slot],
                              sems.at[slot, 0]).start()
        pltpu.make_async_copy(skW2_hbm.at[skill], w2buf.at[slot],
                              sems.at[slot, 1]).start()

    ccopies = [
        pltpu.make_async_copy(concepts_hbm, cbuf, csems.at[0]),
        pltpu.make_async_copy(wq_hbm, wq_s, csems.at[1]),
        pltpu.make_async_copy(wk_hbm, wk_s, csems.at[2]),
        pltpu.make_async_copy(wv_hbm, wv_s, csems.at[3]),
        pltpu.make_async_copy(wo_hbm, wo_s, csems.at[4]),
        pltpu.make_async_copy(keW1_hbm, keW1_s, csems.at[5]),
        pltpu.make_async_copy(keW2_hbm, keW2_s, csems.at[6]),
        pltpu.make_async_copy(fuW1_hbm.at[pl.ds(2 * DIM, 2 * DIM)], fuW1_s,
                              csems.at[7]),
        pltpu.make_async_copy(fuW2_hbm, fuW2_s, csems.at[8]),
    ]

    @pl.when(k == 0)
    def _prologue():
        for j in range(RING):
            _issue(jnp.int32(j), jnp.int32(j))

    _sched = [(0, [0]), (1, [1, 2]), (2, [3, 4]), (3, [5]), (4, [6]),
              (5, [7]), (6, [8])]
    for _step, _idxs in _sched:
        @pl.when(k == _step)
        def _issue_consts(_idxs=_idxs):
            for j in _idxs:
                ccopies[j].start()

    @pl.when(k == 0)
    def _scores():
        selb = pack_ref[_PK_SELB:_PK_SELB + 1, :NS]
        sl = jnp.dot(x, selW_ref[...], preferred_element_type=F32) + selb
        sl = sl - jnp.max(sl, axis=-1, keepdims=True)
        e = jnp.exp(sl)
        sc_ref[...] = e / jnp.sum(e, axis=-1, keepdims=True)

    @pl.when(k == 7)
    def _init():
        for c in ccopies[:7]:
            c.wait()
        bq = pack_ref[_PK_BQ:_PK_BQ + 1, :DIM]
        bv = pack_ref[_PK_BQ:_PK_BQ + 1, DIM:]
        bo = pack_ref[_PK_BO:_PK_BO + 1, :DIM]
        keb2 = pack_ref[_PK_BO:_PK_BO + 1, DIM:]
        q = jnp.dot(x, wq_s[...], preferred_element_type=F32) + bq
        rows = jax.lax.broadcasted_iota(jnp.int32, (H, DIM), 0)
        cols = jax.lax.broadcasted_iota(jnp.int32, (H, DIM), 1)
        maskf = (cols // HD == rows).astype(F32)
        q8 = maskf * q
        t = jax.lax.dot_general(wk_s[...], q8, (((1,), (1,)), ((), ())),
                                preferred_element_type=F32)
        logits = jnp.dot(cbuf[...], t,
                         preferred_element_type=F32) * (1.0 / math.sqrt(HD))
        m = jnp.max(logits, axis=0, keepdims=True)
        ew = jnp.exp(logits - m)
        w = ew / jnp.sum(ew, axis=0, keepdims=True)
        u = jax.lax.dot_general(w, cbuf[...], (((0,), (0,)), ((), ())),
                                preferred_element_type=F32)
        p = jnp.dot(u, wv_s[...], preferred_element_type=F32)
        o = jnp.sum(p * maskf, axis=0, keepdims=True) + bv
        attended = jnp.dot(o, wo_s[...], preferred_element_type=F32) + bo
        combined = x + attended
        keb1 = pack_ref[_PK_KEB1:_PK_KEB1 + 1, :]
        keg1 = pack_ref[_PK_KEG1:_PK_KEG1 + 1, :]
        kebe1 = pack_ref[_PK_KEBE1:_PK_KEBE1 + 1, :]
        y = jnp.dot(combined, keW1_s[...], preferred_element_type=F32) + keb1
        mu = jnp.mean(y, axis=-1, keepdims=True)
        var = jnp.mean((y - mu) * (y - mu), axis=-1, keepdims=True)
        yn = (y - mu) / jnp.sqrt(var + 1e-5) * keg1 + kebe1
        h1 = jnp.maximum(yn, 0.0)
        sem_ref[...] = jnp.dot(h1, keW2_s[...], preferred_element_type=F32) + keb2

    s_all = sc_ref[...]
    lane = jax.lax.broadcasted_iota(jnp.int32, (1, NS), 1)
    base = jax.lax.rem(k, RING // SPB) * SPB
    contrib = None
    for i in range(SPB):
        skill = SPB * k + i
        slot = base + i
        pltpu.make_async_copy(skW1_hbm.at[skill], w1buf.at[slot],
                              sems.at[slot, 0]).wait()
        pltpu.make_async_copy(skW2_hbm.at[skill], w2buf.at[slot],
                              sems.at[slot, 1]).wait()
        sk = jnp.sum(jnp.where(lane == skill, s_all, 0.0))
        b1 = skb_ref[pl.ds(skill, 1), :]
        b2 = skb_ref[pl.ds(NS + skill, 1), :]
        hk = jnp.maximum(
            jnp.dot(x, w1buf[slot], preferred_element_type=F32) + b1, 0.0)
        outk = jnp.dot(hk, w2buf[slot], preferred_element_type=F32) + b2
        c = sk * outk
        contrib = c if contrib is None else contrib + c

    @pl.when(k == 0)
    def _first():
        acc_ref[...] = contrib

    @pl.when(k > 0)
    def _rest():
        acc_ref[...] = acc_ref[...] + contrib

    @pl.when(SPB * k + RING < NS)
    def _refill():
        for i in range(SPB):
            _issue(SPB * k + RING + i, base + i)

    @pl.when(k == NS // SPB - 1)
    def _fuse():
        for c in ccopies[7:]:
            c.wait()
        fub1 = pack_ref[_PK_FUB1:_PK_FUB1 + 1, :]
        fug = pack_ref[_PK_FUG:_PK_FUG + 1, :]
        fube = pack_ref[_PK_FUBE:_PK_FUBE + 1, :]
        fub2 = pack_ref[_PK_SELB:_PK_SELB + 1, DIM:]
        cat = jnp.concatenate([sem_ref[...], acc_ref[...]], axis=-1)
        y = jnp.dot(cat, fuW1_s[...], preferred_element_type=F32) + fub1
        mu = jnp.mean(y, axis=-1, keepdims=True)
        var = jnp.mean((y - mu) * (y - mu), axis=-1, keepdims=True)
        yn = (y - mu) / jnp.sqrt(var + 1e-5) * fug + fube
        fh = jnp.maximum(yn, 0.0)
        out_ref[...] = jnp.dot(fh, fuW2_s[...], preferred_element_type=F32) + fub2


def _const2d(shape):
    return pl.BlockSpec(shape, lambda k: (0, 0))


_HBM = pl.BlockSpec(memory_space=pltpu.MemorySpace.HBM)


@jax.jit
def kernel(x, concepts, Wq, bq, Wk, bk, Wv, bv, Wo, bo, keW1, keb1, keg1,
           kebe1, keW2, keb2, selW, selb, skW1, skb1, skW2, skb2,
           fuW1, fub1, fug, fube, fuW2, fub2):
    d = DIM
    z = jnp.zeros((2 * d,), F32)
    pack = jnp.stack([
        jnp.concatenate([bq, bv]),
        jnp.concatenate([bo, keb2]),
        keb1, keg1, kebe1, fub1, fug, fube,
        jnp.concatenate([selb, z[:d - NS], fub2]),
    ])
    skb = jnp.concatenate([skb1, skb2], axis=0)
    in_specs = [
        _const2d((1, d)),
        _const2d((_PKROWS, 2 * d)),
        _const2d((d, NS)),
        _const2d((2 * NS, d)),
        _HBM, _HBM, _HBM, _HBM, _HBM,
        _HBM, _HBM, _HBM, _HBM,
        _HBM, _HBM,
    ]
    out = pl.pallas_call(
        _body,
        grid=(NS // SPB,),
        in_specs=in_specs,
        out_specs=_const2d((1, d)),
        out_shape=jax.ShapeDtypeStruct((1, d), F32),
        scratch_shapes=[
            pltpu.VMEM((1, d), F32),
            pltpu.VMEM((1, d), F32),
            pltpu.VMEM((1, NS), F32),
            pltpu.VMEM((RING, d, d), F32),
            pltpu.VMEM((RING, d, d), F32),
            pltpu.SemaphoreType.DMA((RING, 2)),
            pltpu.VMEM((NC, d), F32),
            pltpu.VMEM((d, d), F32),
            pltpu.VMEM((d, d), F32),
            pltpu.VMEM((d, d), F32),
            pltpu.VMEM((d, d), F32),
            pltpu.VMEM((d, 2 * d), F32),
            pltpu.VMEM((2 * d, d), F32),
            pltpu.VMEM((2 * d, 2 * d), F32),
            pltpu.VMEM((2 * d, d), F32),
            pltpu.SemaphoreType.DMA((9,)),
        ],
        compiler_params=pltpu.CompilerParams(
            dimension_semantics=("arbitrary",),
            vmem_limit_bytes=67108864,
        ),
    )(x, pack, selW, skb,
      concepts, Wq, Wk, Wv, Wo, keW1, keW2, fuW1, fuW2, skW1, skW2)
    return out
